# Initial kernel scaffold; baseline (speedup 1.0000x reference)
import jax
import jax.numpy as jnp
from jax.experimental import pallas as pl


def kernel(token_ids, table, prefix):
    raise NotImplementedError("write your pallas kernel here")


# speedup vs baseline: 1.2636x; 1.2636x over previous
import functools

import jax
import jax.numpy as jnp
from jax import lax
from jax.experimental import pallas as pl
from jax.experimental.pallas import tpu as pltpu
from jax.experimental.pallas import tpu_sc as plsc

N_CLS = 1000
MAX_TOK = 67
D = 512
SUB = D // 128
PROMPT_LEN = 5
PREFIX_LEN = 5
HEAD = PREFIX_LEN + PROMPT_LEN
MAX_LEN = 77
NW = 32
CLS_PER_W = 32
TOK_PAD = 72
START = MAX_TOK // 2 - PROMPT_LEN // 2
LANES = 16


def _body(tok_hbm, table_hbm, prefix_hbm, out_hbm, idx_v, tok_v, head_v,
          gsem, osem):
    wid = lax.axis_index("s") * 2 + lax.axis_index("c")
    base = wid * CLS_PER_W
    pltpu.sync_copy(tok_hbm.at[pl.ds(base * TOK_PAD, CLS_PER_W * TOK_PAD)],
                    idx_v)
    pltpu.sync_copy(prefix_hbm, head_v.at[pl.ds(0, PREFIX_LEN)])

    def step(i, carry):
        c = base + i

        @pl.when(c < N_CLS)
        def _():
            pltpu.async_copy(
                table_hbm.at[idx_v.at[pl.ds(i * TOK_PAD, TOK_PAD)]],
                tok_v, gsem,
            ).wait()
            for j in range(D // LANES):
                s, o = j // (128 // LANES), (j % (128 // LANES)) * LANES
                col = pl.ds(o, LANES)
                xs = [tok_v[START + k, s, col] for k in range(PROMPT_LEN)]
                mean = (xs[0] + xs[1] + xs[2] + xs[3] + xs[4]) * 0.2
                dfs = [x - mean for x in xs]
                var = (dfs[0] * dfs[0] + dfs[1] * dfs[1] + dfs[2] * dfs[2]
                       + dfs[3] * dfs[3] + dfs[4] * dfs[4]) * 0.25
                yi = jnp.int32(0x5F3759DF) - (
                    lax.bitcast_convert_type(var, jnp.int32) >> 1)
                y = lax.bitcast_convert_type(yi, jnp.float32)
                y = y * (1.5 - 0.5 * var * y * y)
                y = y * (1.5 - 0.5 * var * y * y)
                y = y * (1.5 - 0.5 * var * y * y)
                std = var * y
                scale = 1.0 / (std + 1e-6)
                for k in range(PROMPT_LEN):
                    head_v[PREFIX_LEN + k, s, col] = dfs[k] * scale
            pltpu.async_copy(
                head_v.at[pl.ds(0, HEAD)], out_hbm.at[c, pl.ds(0, HEAD)],
                osem,
            ).wait()
            pltpu.async_copy(
                tok_v.at[pl.ds(0, MAX_TOK)],
                out_hbm.at[c, pl.ds(HEAD, MAX_TOK)], osem,
            ).wait()

        return carry

    lax.fori_loop(0, CLS_PER_W, step, 0)


_sc_call = functools.partial(
    pl.kernel,
    out_type=jax.ShapeDtypeStruct((N_CLS, MAX_LEN, SUB, 128), jnp.float32),
    mesh=plsc.VectorSubcoreMesh(core_axis_name="c", subcore_axis_name="s"),
    scratch_types=[
        pltpu.VMEM((CLS_PER_W * TOK_PAD,), jnp.int32),
        pltpu.VMEM((TOK_PAD, SUB, 128), jnp.float32),
        pltpu.VMEM((16, SUB, 128), jnp.float32),
        pltpu.SemaphoreType.DMA,
        pltpu.SemaphoreType.DMA,
    ],
)(_body)


def kernel(token_ids, table, prefix):
    tok_p = jnp.zeros((NW * CLS_PER_W, TOK_PAD), jnp.int32)
    tok_p = tok_p.at[:N_CLS, :MAX_TOK].set(token_ids.astype(jnp.int32))
    out = _sc_call(tok_p.reshape(-1),
                   table.reshape(table.shape[0], SUB, 128),
                   prefix.reshape(PREFIX_LEN, SUB, 128))
    return out.reshape(N_CLS, MAX_LEN, D)
